# Initial kernel scaffold; baseline (speedup 1.0000x reference)
import jax
import jax.numpy as jnp
from jax.experimental import pallas as pl


def kernel(kind_ids, other_feats, edge_index, cand_u, cand_v, kind_table, W0, b0, W1, b1, W2, b2, Wg, bg, Wc1, bc1, Wc2, bc2):
    raise NotImplementedError("write your pallas kernel here")


# speedup vs baseline: 6.7792x; 6.7792x over previous
import functools

import jax
import jax.numpy as jnp
from jax import lax
from jax.experimental import pallas as pl
from jax.experimental.pallas import tpu as pltpu
from jax.experimental.pallas import tpu_sc as plsc

N = 50000
E = 800000
C = 4096
H = 64

NC = 2
NS = 16
HALF = 25088
NPAD = 2 * HALF
ROWS_PT = HALF // NS
CHUNK = 128
EPT = 50048
EPAD = EPT * NS
NCHUNK = EPT // CHUNK
ZTAIL = ROWS_PT - (ROWS_PT // CHUNK) * CHUNK
BLK = 1024
NBLK = NPAD // BLK

_mesh = plsc.VectorSubcoreMesh(core_axis_name="c", subcore_axis_name="s")


def _fill_const(ref, rows, width, value):
    def body(r, _):
        for j in range(width // 16):
            ref[r, pl.ds(j * 16, 16)] = jnp.full((16,), value, jnp.float32)
        return 0
    lax.fori_loop(0, rows, body, 0)


def _local_idx(idx_d, lidx, base):
    for j in range(CHUNK // 16):
        d = idx_d[pl.ds(j * 16, 16)]
        l = d - base
        bad = (l < 0) | (l >= HALF)
        lidx[pl.ds(j * 16, 16)] = jnp.where(bad, HALF, l)


def _zero_acc(acc, zrows, s, width):
    def zbody(k, _):
        pltpu.sync_copy(zrows, acc.at[pl.ds(s * ROWS_PT + k * CHUNK, CHUNK)])
        return 0
    lax.fori_loop(0, ROWS_PT // CHUNK, zbody, 0)
    pltpu.sync_copy(zrows.at[pl.ds(0, ZTAIL)],
                    acc.at[pl.ds(s * ROWS_PT + (ROWS_PT // CHUNK) * CHUNK, ZTAIL)])

    @pl.when(s == 0)
    def _():
        pltpu.sync_copy(zrows.at[pl.ds(0, 8)], acc.at[pl.ds(HALF, 8)])


@functools.partial(
    pl.kernel,
    out_type=jax.ShapeDtypeStruct((NPAD, 16), jnp.float32),
    mesh=_mesh,
    compiler_params=pltpu.CompilerParams(use_tc_tiling_on_sc=False),
    scratch_types=[
        pltpu.VMEM_SHARED((HALF + 8, 16), jnp.float32),
        pltpu.VMEM((CHUNK,), jnp.int32),
        pltpu.VMEM((CHUNK,), jnp.int32),
        pltpu.VMEM((CHUNK, 16), jnp.float32),
    ],
)
def _deg_kernel(dst_hbm, out_hbm, acc, idx_d, lidx, ones):
    c = lax.axis_index("c")
    s = lax.axis_index("s")
    base = c * HALF
    _fill_const(ones, CHUNK, 16, 0.0)
    _zero_acc(acc, ones, s, 16)
    _fill_const(ones, CHUNK, 16, 1.0)
    plsc.subcore_barrier()

    def ebody(i, _):
        eb = s * EPT + i * CHUNK
        pltpu.sync_copy(dst_hbm.at[pl.ds(eb, CHUNK)], idx_d)
        _local_idx(idx_d, lidx, base)
        pltpu.sync_copy(ones, acc.at[lidx], add=True)
        return 0
    lax.fori_loop(0, NCHUNK, ebody, 0)
    plsc.subcore_barrier()
    pltpu.sync_copy(acc.at[pl.ds(s * ROWS_PT, ROWS_PT)],
                    out_hbm.at[pl.ds(base + s * ROWS_PT, ROWS_PT)])


@functools.partial(
    pl.kernel,
    out_type=jax.ShapeDtypeStruct((NPAD, H), jnp.float32),
    mesh=_mesh,
    compiler_params=pltpu.CompilerParams(use_tc_tiling_on_sc=False),
    scratch_types=[
        pltpu.VMEM_SHARED((HALF + 8, H), jnp.float32),
        pltpu.VMEM((CHUNK,), jnp.int32),
        pltpu.VMEM((CHUNK,), jnp.int32),
        pltpu.VMEM((CHUNK,), jnp.int32),
        pltpu.VMEM((CHUNK, H), jnp.float32),
        pltpu.VMEM((CHUNK, H), jnp.float32),
        pltpu.SemaphoreType.DMA,
    ],
)
def _edge_kernel(xn_hbm, src_hbm, dst_hbm, out_hbm,
                 acc, idx_s, idx_d, lidx, rows, zrows, sem):
    c = lax.axis_index("c")
    s = lax.axis_index("s")
    base = c * HALF
    _fill_const(zrows, CHUNK, H, 0.0)
    _zero_acc(acc, zrows, s, H)
    plsc.subcore_barrier()

    def ebody(i, _):
        eb = s * EPT + i * CHUNK
        pltpu.sync_copy(src_hbm.at[pl.ds(eb, CHUNK)], idx_s)
        pltpu.sync_copy(dst_hbm.at[pl.ds(eb, CHUNK)], idx_d)
        pltpu.async_copy(xn_hbm.at[idx_s], rows, sem).wait()
        _local_idx(idx_d, lidx, base)
        pltpu.sync_copy(rows, acc.at[lidx], add=True)
        return 0
    lax.fori_loop(0, NCHUNK, ebody, 0)
    plsc.subcore_barrier()
    pltpu.sync_copy(acc.at[pl.ds(s * ROWS_PT, ROWS_PT)],
                    out_hbm.at[pl.ds(base + s * ROWS_PT, ROWS_PT)])


@functools.partial(
    pl.kernel,
    out_type=(jax.ShapeDtypeStruct((C, H), jnp.float32),
              jax.ShapeDtypeStruct((C, H), jnp.float32)),
    mesh=_mesh,
    compiler_params=pltpu.CompilerParams(use_tc_tiling_on_sc=False),
    scratch_types=[
        pltpu.VMEM((CHUNK,), jnp.int32),
        pltpu.VMEM((CHUNK, H), jnp.float32),
        pltpu.SemaphoreType.DMA,
    ],
)
def _cand_kernel(h_hbm, cu_hbm, cv_hbm, u_out, v_out, idx, rows, sem):
    c = lax.axis_index("c")
    s = lax.axis_index("s")
    b = (s * NC + c) * CHUNK
    pltpu.sync_copy(cu_hbm.at[pl.ds(b, CHUNK)], idx)
    pltpu.async_copy(h_hbm.at[idx], rows, sem).wait()
    pltpu.sync_copy(rows, u_out.at[pl.ds(b, CHUNK)])
    pltpu.sync_copy(cv_hbm.at[pl.ds(b, CHUNK)], idx)
    pltpu.async_copy(h_hbm.at[idx], rows, sem).wait()
    pltpu.sync_copy(rows, v_out.at[pl.ds(b, CHUNK)])


def _a0_body(kid_ref, of_ref, deg_ref, kt_ref, w0_ref, out_ref):
    norm = lax.rsqrt(deg_ref[:, 0:1] + 1.0)
    iot = lax.broadcasted_iota(jnp.int32, (1, 8), 1)
    onehot = jnp.where(kid_ref[...] == iot, 1.0, 0.0)
    t0 = jnp.dot(kt_ref[...], w0_ref[0:8, :], preferred_element_type=jnp.float32)
    x = (jnp.dot(onehot[:, 0:6], t0, preferred_element_type=jnp.float32)
         + jnp.dot(of_ref[...], w0_ref[8:13, :], preferred_element_type=jnp.float32))
    out_ref[...] = x * norm


def _al_body(s_ref, xn_ref, deg_ref, b_ref, w_ref, out_ref):
    norm = lax.rsqrt(deg_ref[:, 0:1] + 1.0)
    h = jnp.maximum(norm * (s_ref[...] + xn_ref[...]) + b_ref[...], 0.0)
    out_ref[...] = jnp.dot(h, w_ref[...], preferred_element_type=jnp.float32) * norm


def _a3_body(s_ref, xn_ref, deg_ref, b_ref, h_out, psum_out):
    i = pl.program_id(0)
    norm = lax.rsqrt(deg_ref[:, 0:1] + 1.0)
    h = jnp.maximum(norm * (s_ref[...] + xn_ref[...]) + b_ref[...], 0.0)
    h_out[...] = h
    row = i * BLK + lax.broadcasted_iota(jnp.int32, (BLK, 1), 0)
    psum_out[...] = jnp.sum(jnp.where(row < N, h, 0.0), axis=0,
                            keepdims=True)[None, :, :]


def _f_body(ps_ref, wg_ref, bg_ref, u_ref, v_ref, wc1_ref, bc1_ref,
            wc2_ref, bc2_ref, out_ref):
    hsum = jnp.sum(ps_ref[...], axis=0)
    g = jnp.dot(hsum * (1.0 / N), wg_ref[...],
                preferred_element_type=jnp.float32) + bg_ref[...]
    cvec = jnp.dot(g, wc1_ref[2 * H:3 * H, :],
                   preferred_element_type=jnp.float32) + bc1_ref[...]
    hid = jnp.maximum(
        jnp.dot(u_ref[...], wc1_ref[0:H, :], preferred_element_type=jnp.float32)
        + jnp.dot(v_ref[...], wc1_ref[H:2 * H, :], preferred_element_type=jnp.float32)
        + cvec, 0.0)
    out_ref[...] = jnp.dot(hid, wc2_ref[...],
                           preferred_element_type=jnp.float32) + bc2_ref[...]


_a0_call = pl.pallas_call(
    _a0_body,
    grid=(NBLK,),
    in_specs=[
        pl.BlockSpec((BLK, 1), lambda i: (i, 0)),
        pl.BlockSpec((BLK, 5), lambda i: (i, 0)),
        pl.BlockSpec((BLK, 16), lambda i: (i, 0)),
        pl.BlockSpec((6, 8), lambda i: (0, 0)),
        pl.BlockSpec((13, H), lambda i: (0, 0)),
    ],
    out_specs=pl.BlockSpec((BLK, H), lambda i: (i, 0)),
    out_shape=jax.ShapeDtypeStruct((NPAD, H), jnp.float32),
)

_al_call = pl.pallas_call(
    _al_body,
    grid=(NBLK,),
    in_specs=[
        pl.BlockSpec((BLK, H), lambda i: (i, 0)),
        pl.BlockSpec((BLK, H), lambda i: (i, 0)),
        pl.BlockSpec((BLK, 16), lambda i: (i, 0)),
        pl.BlockSpec((1, H), lambda i: (0, 0)),
        pl.BlockSpec((H, H), lambda i: (0, 0)),
    ],
    out_specs=pl.BlockSpec((BLK, H), lambda i: (i, 0)),
    out_shape=jax.ShapeDtypeStruct((NPAD, H), jnp.float32),
)

_a3_call = pl.pallas_call(
    _a3_body,
    grid=(NBLK,),
    in_specs=[
        pl.BlockSpec((BLK, H), lambda i: (i, 0)),
        pl.BlockSpec((BLK, H), lambda i: (i, 0)),
        pl.BlockSpec((BLK, 16), lambda i: (i, 0)),
        pl.BlockSpec((1, H), lambda i: (0, 0)),
    ],
    out_specs=(pl.BlockSpec((BLK, H), lambda i: (i, 0)),
               pl.BlockSpec((1, 1, H), lambda i: (i, 0, 0))),
    out_shape=(jax.ShapeDtypeStruct((NPAD, H), jnp.float32),
               jax.ShapeDtypeStruct((NBLK, 1, H), jnp.float32)),
)

_f_call = pl.pallas_call(
    _f_body,
    out_shape=jax.ShapeDtypeStruct((C, 1), jnp.float32),
)


def kernel(kind_ids, other_feats, edge_index, cand_u, cand_v, kind_table,
           W0, b0, W1, b1, W2, b2, Wg, bg, Wc1, bc1, Wc2, bc2):
    src = edge_index[0].astype(jnp.int32)
    dst = edge_index[1].astype(jnp.int32)
    srcp = jnp.concatenate([src, jnp.zeros((EPAD - E,), jnp.int32)])
    dstp = jnp.concatenate([dst, jnp.full((EPAD - E,), NPAD, jnp.int32)])
    kidp = jnp.concatenate([kind_ids.astype(jnp.int32),
                            jnp.zeros((NPAD - N,), jnp.int32)]).reshape(NPAD, 1)
    ofp = jnp.concatenate([other_feats,
                           jnp.zeros((NPAD - N, 5), jnp.float32)])

    deg16 = _deg_kernel(dstp)
    xn0 = _a0_call(kidp, ofp, deg16, kind_table, W0)
    s0 = _edge_kernel(xn0, srcp, dstp)
    xn1 = _al_call(s0, xn0, deg16, b0.reshape(1, H), W1)
    s1 = _edge_kernel(xn1, srcp, dstp)
    xn2 = _al_call(s1, xn1, deg16, b1.reshape(1, H), W2)
    s2 = _edge_kernel(xn2, srcp, dstp)
    h3, psum = _a3_call(s2, xn2, deg16, b2.reshape(1, H))
    u_emb, v_emb = _cand_kernel(h3, cand_u.astype(jnp.int32),
                                cand_v.astype(jnp.int32))
    logits = _f_call(psum, Wg, bg.reshape(1, H), u_emb, v_emb,
                     Wc1, bc1.reshape(1, H), Wc2, bc2.reshape(1, 1))
    return logits[:, 0]


# speedup vs baseline: 8.3419x; 1.2305x over previous
import functools

import jax
import jax.numpy as jnp
from jax import lax
from jax.experimental import pallas as pl
from jax.experimental.pallas import tpu as pltpu
from jax.experimental.pallas import tpu_sc as plsc

N = 50000
E = 800000
C = 4096
H = 64

NC = 2
NS = 16
HALF = 25008
NPAD = 50176
ROWS_PT = HALF // NS
CHUNK = 128
K = 1
GROUP = K * CHUNK
NB = 3
NG = 393
EPT = NG * GROUP
EPAD = EPT * NS
EROWS_PT = EPT // CHUNK
ZTAIL = ROWS_PT - (ROWS_PT // CHUNK) * CHUNK
BLK = 1024
NBLK = NPAD // BLK

_mesh = plsc.VectorSubcoreMesh(core_axis_name="c", subcore_axis_name="s")


def _fill_const(ref, rows, width, value):
    def body(r, _):
        for j in range(width // 16):
            ref[r, pl.ds(j * 16, 16)] = jnp.full((16,), value, jnp.float32)
        return 0
    lax.fori_loop(0, rows, body, 0)


def _zero_acc(acc, zsrc, s):
    def zbody(k, _):
        pltpu.sync_copy(zsrc, acc.at[pl.ds(s * ROWS_PT + k * CHUNK, CHUNK)])
        return 0
    lax.fori_loop(0, ROWS_PT // CHUNK, zbody, 0)
    pltpu.sync_copy(zsrc.at[pl.ds(0, ZTAIL)],
                    acc.at[pl.ds(s * ROWS_PT + (ROWS_PT // CHUNK) * CHUNK, ZTAIL)])

    @pl.when(s == 0)
    def _():
        pltpu.sync_copy(zsrc.at[pl.ds(0, 1)], acc.at[pl.ds(HALF, 1)])


def _compute_lidx(idx_d, lidx, base):
    for k in range(K):
        for j in range(CHUNK // 16):
            d = idx_d[k, pl.ds(j * 16, 16)]
            l = d - base
            bad = (l < 0) | (l >= HALF)
            lidx[k, pl.ds(j * 16, 16)] = jnp.where(bad, HALF, l)


@functools.partial(
    pl.kernel,
    out_type=jax.ShapeDtypeStruct((NPAD, 16), jnp.float32),
    mesh=_mesh,
    compiler_params=pltpu.CompilerParams(use_tc_tiling_on_sc=False),
    scratch_types=[
        pltpu.VMEM_SHARED((HALF + 1, 16), jnp.float32),
        pltpu.VMEM((1, CHUNK), jnp.int32),
        pltpu.VMEM((1, CHUNK), jnp.int32),
        pltpu.VMEM((1, CHUNK), jnp.int32),
        pltpu.VMEM((1, CHUNK), jnp.int32),
        pltpu.VMEM((1, CHUNK), jnp.int32),
        pltpu.VMEM((1, CHUNK), jnp.int32),
        pltpu.VMEM((CHUNK, 16), jnp.float32),
        pltpu.VMEM((CHUNK, 16), jnp.float32),
        pltpu.SemaphoreType.DMA,
        pltpu.SemaphoreType.DMA,
        pltpu.SemaphoreType.DMA,
        pltpu.SemaphoreType.DMA,
        pltpu.SemaphoreType.DMA,
        pltpu.SemaphoreType.DMA,
    ],
)
def _deg_kernel(dst_hbm, out_hbm, acc,
                id0, id1, id2, li0, li1, li2, ones, zr16,
                si0, si1, si2, ss0, ss1, ss2):
    c = lax.axis_index("c")
    s = lax.axis_index("s")
    base = c * HALF
    ID = (id0, id1, id2)
    LI = (li0, li1, li2)
    SI = (si0, si1, si2)
    SS = (ss0, ss1, ss2)

    _fill_const(zr16, CHUNK, 16, 0.0)
    _zero_acc(acc, zr16, s)
    _fill_const(ones, CHUNK, 16, 1.0)
    plsc.subcore_barrier()

    row_base = s * EROWS_PT

    def issue_idx(g, b):
        pltpu.async_copy(dst_hbm.at[pl.ds(row_base + g, 1)], ID[b], SI[b])

    def drain_idx(b):
        pltpu.make_async_copy(dst_hbm.at[pl.ds(0, 1)], ID[b], SI[b]).wait()

    def issue_scatter(b):
        for k in range(K):
            pltpu.async_copy(ones, acc.at[LI[b].at[k]], SS[b], add=True)

    def drain_scatter(b):
        for k in range(K):
            pltpu.make_async_copy(out_hbm.at[pl.ds(0, CHUNK)], ones,
                                  SS[b]).wait()

    issue_idx(0, 0)

    def step(t, _):
        for b in range(NB):
            g = t * NB + b
            if b == NB - 1:
                @pl.when(t < NG // NB - 1)
                def _():
                    issue_idx(g + 1, (b + 1) % NB)
            else:
                issue_idx(g + 1, (b + 1) % NB)
            drain_idx(b)

            @pl.when(t >= 1)
            def _():
                drain_scatter(b)
            _compute_lidx(ID[b], LI[b], base)
            issue_scatter(b)
        return 0
    lax.fori_loop(0, NG // NB, step, 0)
    for b in range(NB):
        drain_scatter(b)
    plsc.subcore_barrier()
    pltpu.sync_copy(acc.at[pl.ds(s * ROWS_PT, ROWS_PT)],
                    out_hbm.at[pl.ds(base + s * ROWS_PT, ROWS_PT)])


@functools.partial(
    pl.kernel,
    out_type=jax.ShapeDtypeStruct((NPAD, H), jnp.float32),
    mesh=_mesh,
    compiler_params=pltpu.CompilerParams(use_tc_tiling_on_sc=False),
    scratch_types=[
        pltpu.VMEM_SHARED((HALF + 1, H), jnp.float32),
        pltpu.VMEM((1, CHUNK), jnp.int32),
        pltpu.VMEM((1, CHUNK), jnp.int32),
        pltpu.VMEM((1, CHUNK), jnp.int32),
        pltpu.VMEM((1, CHUNK), jnp.int32),
        pltpu.VMEM((1, CHUNK), jnp.int32),
        pltpu.VMEM((1, CHUNK), jnp.int32),
        pltpu.VMEM((1, CHUNK), jnp.int32),
        pltpu.VMEM((1, CHUNK), jnp.int32),
        pltpu.VMEM((1, CHUNK), jnp.int32),
        pltpu.VMEM((GROUP, H), jnp.float32),
        pltpu.VMEM((GROUP, H), jnp.float32),
        pltpu.VMEM((GROUP, H), jnp.float32),
        pltpu.SemaphoreType.DMA,
        pltpu.SemaphoreType.DMA,
        pltpu.SemaphoreType.DMA,
        pltpu.SemaphoreType.DMA,
        pltpu.SemaphoreType.DMA,
        pltpu.SemaphoreType.DMA,
        pltpu.SemaphoreType.DMA,
        pltpu.SemaphoreType.DMA,
        pltpu.SemaphoreType.DMA,
    ],
)
def _edge_kernel(xn_hbm, src_hbm, dst_hbm, out_hbm, acc,
                 is0, is1, is2, id0, id1, id2, li0, li1, li2,
                 r0, r1, r2,
                 si0, si1, si2, sg0, sg1, sg2, ss0, ss1, ss2):
    c = lax.axis_index("c")
    s = lax.axis_index("s")
    base = c * HALF
    IS = (is0, is1, is2)
    ID = (id0, id1, id2)
    LI = (li0, li1, li2)
    R = (r0, r1, r2)
    SI = (si0, si1, si2)
    SG = (sg0, sg1, sg2)
    SS = (ss0, ss1, ss2)

    _fill_const(r0, CHUNK, H, 0.0)
    _zero_acc(acc, r0.at[pl.ds(0, CHUNK)], s)
    plsc.subcore_barrier()

    row_base = s * EROWS_PT

    def issue_idx(g, b):
        pltpu.async_copy(src_hbm.at[pl.ds(row_base + g, 1)], IS[b], SI[b])
        pltpu.async_copy(dst_hbm.at[pl.ds(row_base + g, 1)], ID[b], SI[b])

    def drain_idx(b):
        pltpu.make_async_copy(src_hbm.at[pl.ds(0, 1)], IS[b], SI[b]).wait()
        pltpu.make_async_copy(dst_hbm.at[pl.ds(0, 1)], ID[b], SI[b]).wait()

    def issue_gather(b):
        for k in range(K):
            pltpu.async_copy(xn_hbm.at[IS[b].at[k]],
                             R[b].at[pl.ds(k * CHUNK, CHUNK)], SG[b])

    def drain_gather(b):
        pltpu.make_async_copy(xn_hbm.at[pl.ds(0, GROUP)], R[b], SG[b]).wait()

    def issue_scatter(b):
        for k in range(K):
            pltpu.async_copy(R[b].at[pl.ds(k * CHUNK, CHUNK)],
                             acc.at[LI[b].at[k]], SS[b], add=True)

    def drain_scatter(b):
        pltpu.make_async_copy(xn_hbm.at[pl.ds(0, GROUP)], R[b], SS[b]).wait()

    issue_idx(0, 0)

    def step(t, _):
        for b in range(NB):
            g = t * NB + b
            bp = (b + NB - 1) % NB
            if b == NB - 1:
                @pl.when(t < NG // NB - 1)
                def _():
                    issue_idx(g + 1, (b + 1) % NB)
            else:
                issue_idx(g + 1, (b + 1) % NB)
            drain_idx(b)

            @pl.when(t >= 1)
            def _():
                drain_scatter(b)
            issue_gather(b)
            _compute_lidx(ID[b], LI[b], base)
            if b == 0:
                @pl.when(t >= 1)
                def _():
                    drain_gather(bp)
                    issue_scatter(bp)
            else:
                drain_gather(bp)
                issue_scatter(bp)
        return 0
    lax.fori_loop(0, NG // NB, step, 0)
    drain_gather((NG - 1) % NB)
    issue_scatter((NG - 1) % NB)
    for b in range(NB):
        drain_scatter(b)
    plsc.subcore_barrier()
    pltpu.sync_copy(acc.at[pl.ds(s * ROWS_PT, ROWS_PT)],
                    out_hbm.at[pl.ds(base + s * ROWS_PT, ROWS_PT)])


@functools.partial(
    pl.kernel,
    out_type=(jax.ShapeDtypeStruct((C, H), jnp.float32),
              jax.ShapeDtypeStruct((C, H), jnp.float32)),
    mesh=_mesh,
    compiler_params=pltpu.CompilerParams(use_tc_tiling_on_sc=False),
    scratch_types=[
        pltpu.VMEM((CHUNK,), jnp.int32),
        pltpu.VMEM((CHUNK, H), jnp.float32),
        pltpu.SemaphoreType.DMA,
    ],
)
def _cand_kernel(h_hbm, cu_hbm, cv_hbm, u_out, v_out, idx, rows, sem):
    c = lax.axis_index("c")
    s = lax.axis_index("s")
    b = (s * NC + c) * CHUNK
    pltpu.sync_copy(cu_hbm.at[pl.ds(b, CHUNK)], idx)
    pltpu.async_copy(h_hbm.at[idx], rows, sem).wait()
    pltpu.sync_copy(rows, u_out.at[pl.ds(b, CHUNK)])
    pltpu.sync_copy(cv_hbm.at[pl.ds(b, CHUNK)], idx)
    pltpu.async_copy(h_hbm.at[idx], rows, sem).wait()
    pltpu.sync_copy(rows, v_out.at[pl.ds(b, CHUNK)])


def _a0_body(kid_ref, of_ref, deg_ref, kt_ref, w0_ref, out_ref):
    norm = lax.rsqrt(deg_ref[:, 0:1] + 1.0)
    iot = lax.broadcasted_iota(jnp.int32, (1, 8), 1)
    onehot = jnp.where(kid_ref[...] == iot, 1.0, 0.0)
    t0 = jnp.dot(kt_ref[...], w0_ref[0:8, :], preferred_element_type=jnp.float32)
    x = (jnp.dot(onehot[:, 0:6], t0, preferred_element_type=jnp.float32)
         + jnp.dot(of_ref[...], w0_ref[8:13, :], preferred_element_type=jnp.float32))
    out_ref[...] = x * norm


def _al_body(s_ref, xn_ref, deg_ref, b_ref, w_ref, out_ref):
    norm = lax.rsqrt(deg_ref[:, 0:1] + 1.0)
    h = jnp.maximum(norm * (s_ref[...] + xn_ref[...]) + b_ref[...], 0.0)
    out_ref[...] = jnp.dot(h, w_ref[...], preferred_element_type=jnp.float32) * norm


def _a3_body(s_ref, xn_ref, deg_ref, b_ref, h_out, psum_out):
    i = pl.program_id(0)
    norm = lax.rsqrt(deg_ref[:, 0:1] + 1.0)
    h = jnp.maximum(norm * (s_ref[...] + xn_ref[...]) + b_ref[...], 0.0)
    h_out[...] = h
    row = i * BLK + lax.broadcasted_iota(jnp.int32, (BLK, 1), 0)
    psum_out[...] = jnp.sum(jnp.where(row < N, h, 0.0), axis=0,
                            keepdims=True)[None, :, :]


def _f_body(ps_ref, wg_ref, bg_ref, u_ref, v_ref, wc1_ref, bc1_ref,
            wc2_ref, bc2_ref, out_ref):
    hsum = jnp.sum(ps_ref[...], axis=0)
    g = jnp.dot(hsum * (1.0 / N), wg_ref[...],
                preferred_element_type=jnp.float32) + bg_ref[...]
    cvec = jnp.dot(g, wc1_ref[2 * H:3 * H, :],
                   preferred_element_type=jnp.float32) + bc1_ref[...]
    hid = jnp.maximum(
        jnp.dot(u_ref[...], wc1_ref[0:H, :], preferred_element_type=jnp.float32)
        + jnp.dot(v_ref[...], wc1_ref[H:2 * H, :], preferred_element_type=jnp.float32)
        + cvec, 0.0)
    out_ref[...] = jnp.dot(hid, wc2_ref[...],
                           preferred_element_type=jnp.float32) + bc2_ref[...]


_a0_call = pl.pallas_call(
    _a0_body,
    grid=(NBLK,),
    in_specs=[
        pl.BlockSpec((BLK, 1), lambda i: (i, 0)),
        pl.BlockSpec((BLK, 5), lambda i: (i, 0)),
        pl.BlockSpec((BLK, 16), lambda i: (i, 0)),
        pl.BlockSpec((6, 8), lambda i: (0, 0)),
        pl.BlockSpec((13, H), lambda i: (0, 0)),
    ],
    out_specs=pl.BlockSpec((BLK, H), lambda i: (i, 0)),
    out_shape=jax.ShapeDtypeStruct((NPAD, H), jnp.float32),
)

_al_call = pl.pallas_call(
    _al_body,
    grid=(NBLK,),
    in_specs=[
        pl.BlockSpec((BLK, H), lambda i: (i, 0)),
        pl.BlockSpec((BLK, H), lambda i: (i, 0)),
        pl.BlockSpec((BLK, 16), lambda i: (i, 0)),
        pl.BlockSpec((1, H), lambda i: (0, 0)),
        pl.BlockSpec((H, H), lambda i: (0, 0)),
    ],
    out_specs=pl.BlockSpec((BLK, H), lambda i: (i, 0)),
    out_shape=jax.ShapeDtypeStruct((NPAD, H), jnp.float32),
)

_a3_call = pl.pallas_call(
    _a3_body,
    grid=(NBLK,),
    in_specs=[
        pl.BlockSpec((BLK, H), lambda i: (i, 0)),
        pl.BlockSpec((BLK, H), lambda i: (i, 0)),
        pl.BlockSpec((BLK, 16), lambda i: (i, 0)),
        pl.BlockSpec((1, H), lambda i: (0, 0)),
    ],
    out_specs=(pl.BlockSpec((BLK, H), lambda i: (i, 0)),
               pl.BlockSpec((1, 1, H), lambda i: (i, 0, 0))),
    out_shape=(jax.ShapeDtypeStruct((NPAD, H), jnp.float32),
               jax.ShapeDtypeStruct((NBLK, 1, H), jnp.float32)),
)

_f_call = pl.pallas_call(
    _f_body,
    out_shape=jax.ShapeDtypeStruct((C, 1), jnp.float32),
)


def kernel(kind_ids, other_feats, edge_index, cand_u, cand_v, kind_table,
           W0, b0, W1, b1, W2, b2, Wg, bg, Wc1, bc1, Wc2, bc2):
    src = edge_index[0].astype(jnp.int32)
    dst = edge_index[1].astype(jnp.int32)
    srcp = jnp.concatenate([src, jnp.zeros((EPAD - E,), jnp.int32)])
    dstp = jnp.concatenate([dst, jnp.full((EPAD - E,), NPAD, jnp.int32)])
    src2d = srcp.reshape(EPAD // CHUNK, CHUNK)
    dst2d = dstp.reshape(EPAD // CHUNK, CHUNK)
    kidp = jnp.concatenate([kind_ids.astype(jnp.int32),
                            jnp.zeros((NPAD - N,), jnp.int32)]).reshape(NPAD, 1)
    ofp = jnp.concatenate([other_feats,
                           jnp.zeros((NPAD - N, 5), jnp.float32)])

    deg16 = _deg_kernel(dst2d)
    xn0 = _a0_call(kidp, ofp, deg16, kind_table, W0)
    s0 = _edge_kernel(xn0, src2d, dst2d)
    xn1 = _al_call(s0, xn0, deg16, b0.reshape(1, H), W1)
    s1 = _edge_kernel(xn1, src2d, dst2d)
    xn2 = _al_call(s1, xn1, deg16, b1.reshape(1, H), W2)
    s2 = _edge_kernel(xn2, src2d, dst2d)
    h3, psum = _a3_call(s2, xn2, deg16, b2.reshape(1, H))
    u_emb, v_emb = _cand_kernel(h3, cand_u.astype(jnp.int32),
                                cand_v.astype(jnp.int32))
    logits = _f_call(psum, Wg, bg.reshape(1, H), u_emb, v_emb,
                     Wc1, bc1.reshape(1, H), Wc2, bc2.reshape(1, 1))
    return logits[:, 0]


# speedup vs baseline: 8.7660x; 1.0508x over previous
import functools

import jax
import jax.numpy as jnp
from jax import lax
from jax.experimental import pallas as pl
from jax.experimental.pallas import tpu as pltpu
from jax.experimental.pallas import tpu_sc as plsc

N = 50000
E = 800000
C = 4096
H = 64

NC = 2
NS = 16
HALF = 25008
NPAD = 50176
ROWS_PT = HALF // NS
CHUNK = 128
IDXN = 112
K = 2
GROUP = K * IDXN
NB = 2
NG = 224
EPT = NG * GROUP
EPAD = EPT * NS
ZTAIL = ROWS_PT - (ROWS_PT // CHUNK) * CHUNK
BLK = 1024
NBLK = NPAD // BLK

_mesh = plsc.VectorSubcoreMesh(core_axis_name="c", subcore_axis_name="s")


def _fill_const(ref, rows, width, value):
    def body(r, _):
        for j in range(width // 16):
            ref[r, pl.ds(j * 16, 16)] = jnp.full((16,), value, jnp.float32)
        return 0
    lax.fori_loop(0, rows, body, 0)


def _zero_acc(acc, zsrc, s):
    def zbody(k, _):
        pltpu.sync_copy(zsrc, acc.at[pl.ds(s * ROWS_PT + k * CHUNK, CHUNK)])
        return 0
    lax.fori_loop(0, ROWS_PT // CHUNK, zbody, 0)
    pltpu.sync_copy(zsrc.at[pl.ds(0, ZTAIL)],
                    acc.at[pl.ds(s * ROWS_PT + (ROWS_PT // CHUNK) * CHUNK, ZTAIL)])

    @pl.when(s == 0)
    def _():
        pltpu.sync_copy(zsrc.at[pl.ds(0, 1)], acc.at[pl.ds(HALF, 1)])


def _compute_lidx(eb, lidx, base):
    for k in range(K):
        for j in range(IDXN // 16):
            d = eb[k, 1, pl.ds(j * 16, 16)]
            l = d - base
            bad = (l < 0) | (l >= HALF)
            lidx[k, pl.ds(j * 16, 16)] = jnp.where(bad, HALF, l)


@functools.partial(
    pl.kernel,
    out_type=jax.ShapeDtypeStruct((NPAD, 16), jnp.float32),
    mesh=_mesh,
    compiler_params=pltpu.CompilerParams(use_tc_tiling_on_sc=False),
    scratch_types=[
        pltpu.VMEM_SHARED((HALF + 1, 16), jnp.float32),
        pltpu.VMEM((K, 2, IDXN), jnp.int32),
        pltpu.VMEM((K, 2, IDXN), jnp.int32),
        pltpu.VMEM((K, IDXN), jnp.int32),
        pltpu.VMEM((K, IDXN), jnp.int32),
        pltpu.VMEM((IDXN, 16), jnp.float32),
        pltpu.VMEM((CHUNK, 16), jnp.float32),
        pltpu.SemaphoreType.DMA,
        pltpu.SemaphoreType.DMA,
        pltpu.SemaphoreType.DMA,
        pltpu.SemaphoreType.DMA,
    ],
)
def _deg_kernel(e_hbm, out_hbm, acc,
                eb0, eb1, li0, li1, ones, zr16,
                si0, si1, ss0, ss1):
    c = lax.axis_index("c")
    s = lax.axis_index("s")
    base = c * HALF
    EB = (eb0, eb1)
    LI = (li0, li1)
    SI = (si0, si1)
    SS = (ss0, ss1)

    _fill_const(zr16, CHUNK, 16, 0.0)
    _zero_acc(acc, zr16, s)
    _fill_const(ones, IDXN, 16, 1.0)
    plsc.subcore_barrier()

    row_base = s * (K * NG)

    def issue_idx(g, b):
        pltpu.async_copy(e_hbm.at[pl.ds(row_base + K * g, K)], EB[b], SI[b])

    def drain_idx(b):
        pltpu.make_async_copy(e_hbm.at[pl.ds(0, K)], EB[b], SI[b]).wait()

    def issue_scatter(b):
        for k in range(K):
            pltpu.async_copy(ones, acc.at[LI[b].at[k]], SS[b], add=True)

    def drain_scatter(b):
        for k in range(K):
            pltpu.make_async_copy(out_hbm.at[pl.ds(0, IDXN)], ones,
                                  SS[b]).wait()

    issue_idx(0, 0)

    def step(t, _):
        for b in range(NB):
            g = t * NB + b
            bp = 1 - b
            if b == NB - 1:
                @pl.when(t < NG // NB - 1)
                def _():
                    issue_idx(g + 1, bp)
            else:
                issue_idx(g + 1, bp)
            drain_idx(b)

            @pl.when(t >= 1)
            def _():
                drain_scatter(b)
            _compute_lidx(EB[b], LI[b], base)
            issue_scatter(b)
        return 0
    lax.fori_loop(0, NG // NB, step, 0)
    for b in range(NB):
        drain_scatter(b)
    plsc.subcore_barrier()
    pltpu.sync_copy(acc.at[pl.ds(s * ROWS_PT, ROWS_PT)],
                    out_hbm.at[pl.ds(base + s * ROWS_PT, ROWS_PT)])


@functools.partial(
    pl.kernel,
    out_type=jax.ShapeDtypeStruct((NPAD, H), jnp.float32),
    mesh=_mesh,
    compiler_params=pltpu.CompilerParams(use_tc_tiling_on_sc=False),
    scratch_types=[
        pltpu.VMEM_SHARED((HALF + 1, H), jnp.float32),
        pltpu.VMEM((K, 2, IDXN), jnp.int32),
        pltpu.VMEM((K, 2, IDXN), jnp.int32),
        pltpu.VMEM((K, IDXN), jnp.int32),
        pltpu.VMEM((K, IDXN), jnp.int32),
        pltpu.VMEM((GROUP, H), jnp.float32),
        pltpu.VMEM((GROUP, H), jnp.float32),
        pltpu.SemaphoreType.DMA,
        pltpu.SemaphoreType.DMA,
        pltpu.SemaphoreType.DMA,
        pltpu.SemaphoreType.DMA,
        pltpu.SemaphoreType.DMA,
        pltpu.SemaphoreType.DMA,
    ],
)
def _edge_kernel(xn_hbm, e_hbm, out_hbm, acc,
                 eb0, eb1, li0, li1, r0, r1,
                 si0, si1, sg0, sg1, ss0, ss1):
    c = lax.axis_index("c")
    s = lax.axis_index("s")
    base = c * HALF
    EB = (eb0, eb1)
    LI = (li0, li1)
    R = (r0, r1)
    SI = (si0, si1)
    SG = (sg0, sg1)
    SS = (ss0, ss1)

    _fill_const(r0, CHUNK, H, 0.0)
    _zero_acc(acc, r0.at[pl.ds(0, CHUNK)], s)
    plsc.subcore_barrier()

    row_base = s * (K * NG)

    def issue_idx(g, b):
        pltpu.async_copy(e_hbm.at[pl.ds(row_base + K * g, K)], EB[b], SI[b])

    def drain_idx(b):
        pltpu.make_async_copy(e_hbm.at[pl.ds(0, K)], EB[b], SI[b]).wait()

    def issue_gather(b):
        for k in range(K):
            pltpu.async_copy(xn_hbm.at[EB[b].at[k, 0]],
                             R[b].at[pl.ds(k * IDXN, IDXN)], SG[b])

    def drain_gather(b):
        pltpu.make_async_copy(xn_hbm.at[pl.ds(0, GROUP)], R[b], SG[b]).wait()

    def issue_scatter(b):
        for k in range(K):
            pltpu.async_copy(R[b].at[pl.ds(k * IDXN, IDXN)],
                             acc.at[LI[b].at[k]], SS[b], add=True)

    def drain_scatter(b):
        pltpu.make_async_copy(xn_hbm.at[pl.ds(0, GROUP)], R[b], SS[b]).wait()

    issue_idx(0, 0)

    def step(t, _):
        for b in range(NB):
            g = t * NB + b
            bp = 1 - b
            drain_idx(b)

            @pl.when(t >= 1)
            def _():
                drain_scatter(b)
            issue_gather(b)
            _compute_lidx(EB[b], LI[b], base)
            if b == 0:
                @pl.when(t >= 1)
                def _():
                    drain_gather(bp)
                    issue_scatter(bp)
            else:
                drain_gather(bp)
                issue_scatter(bp)
            if b == NB - 1:
                @pl.when(t < NG // NB - 1)
                def _():
                    issue_idx(g + 1, bp)
            else:
                issue_idx(g + 1, bp)
        return 0
    lax.fori_loop(0, NG // NB, step, 0)
    drain_gather((NG - 1) % NB)
    issue_scatter((NG - 1) % NB)
    for b in range(NB):
        drain_scatter(b)
    plsc.subcore_barrier()
    pltpu.sync_copy(acc.at[pl.ds(s * ROWS_PT, ROWS_PT)],
                    out_hbm.at[pl.ds(base + s * ROWS_PT, ROWS_PT)])


@functools.partial(
    pl.kernel,
    out_type=(jax.ShapeDtypeStruct((C, H), jnp.float32),
              jax.ShapeDtypeStruct((C, H), jnp.float32)),
    mesh=_mesh,
    compiler_params=pltpu.CompilerParams(use_tc_tiling_on_sc=False),
    scratch_types=[
        pltpu.VMEM((CHUNK,), jnp.int32),
        pltpu.VMEM((CHUNK, H), jnp.float32),
        pltpu.SemaphoreType.DMA,
    ],
)
def _cand_kernel(h_hbm, cu_hbm, cv_hbm, u_out, v_out, idx, rows, sem):
    c = lax.axis_index("c")
    s = lax.axis_index("s")
    b = (s * NC + c) * CHUNK
    pltpu.sync_copy(cu_hbm.at[pl.ds(b, CHUNK)], idx)
    pltpu.async_copy(h_hbm.at[idx], rows, sem).wait()
    pltpu.sync_copy(rows, u_out.at[pl.ds(b, CHUNK)])
    pltpu.sync_copy(cv_hbm.at[pl.ds(b, CHUNK)], idx)
    pltpu.async_copy(h_hbm.at[idx], rows, sem).wait()
    pltpu.sync_copy(rows, v_out.at[pl.ds(b, CHUNK)])


def _a0_body(kid_ref, of_ref, deg_ref, kt_ref, w0_ref, out_ref):
    norm = lax.rsqrt(deg_ref[:, 0:1] + 1.0)
    iot = lax.broadcasted_iota(jnp.int32, (1, 8), 1)
    onehot = jnp.where(kid_ref[...] == iot, 1.0, 0.0)
    t0 = jnp.dot(kt_ref[...], w0_ref[0:8, :], preferred_element_type=jnp.float32)
    x = (jnp.dot(onehot[:, 0:6], t0, preferred_element_type=jnp.float32)
         + jnp.dot(of_ref[...], w0_ref[8:13, :], preferred_element_type=jnp.float32))
    out_ref[...] = x * norm


def _al_body(s_ref, xn_ref, deg_ref, b_ref, w_ref, out_ref):
    norm = lax.rsqrt(deg_ref[:, 0:1] + 1.0)
    h = jnp.maximum(norm * (s_ref[...] + xn_ref[...]) + b_ref[...], 0.0)
    out_ref[...] = jnp.dot(h, w_ref[...], preferred_element_type=jnp.float32) * norm


def _a3_body(s_ref, xn_ref, deg_ref, b_ref, h_out, psum_out):
    i = pl.program_id(0)
    norm = lax.rsqrt(deg_ref[:, 0:1] + 1.0)
    h = jnp.maximum(norm * (s_ref[...] + xn_ref[...]) + b_ref[...], 0.0)
    h_out[...] = h
    row = i * BLK + lax.broadcasted_iota(jnp.int32, (BLK, 1), 0)
    psum_out[...] = jnp.sum(jnp.where(row < N, h, 0.0), axis=0,
                            keepdims=True)[None, :, :]


def _f_body(ps_ref, wg_ref, bg_ref, u_ref, v_ref, wc1_ref, bc1_ref,
            wc2_ref, bc2_ref, out_ref):
    hsum = jnp.sum(ps_ref[...], axis=0)
    g = jnp.dot(hsum * (1.0 / N), wg_ref[...],
                preferred_element_type=jnp.float32) + bg_ref[...]
    cvec = jnp.dot(g, wc1_ref[2 * H:3 * H, :],
                   preferred_element_type=jnp.float32) + bc1_ref[...]
    hid = jnp.maximum(
        jnp.dot(u_ref[...], wc1_ref[0:H, :], preferred_element_type=jnp.float32)
        + jnp.dot(v_ref[...], wc1_ref[H:2 * H, :], preferred_element_type=jnp.float32)
        + cvec, 0.0)
    out_ref[...] = jnp.dot(hid, wc2_ref[...],
                           preferred_element_type=jnp.float32) + bc2_ref[...]


_a0_call = pl.pallas_call(
    _a0_body,
    grid=(NBLK,),
    in_specs=[
        pl.BlockSpec((BLK, 1), lambda i: (i, 0)),
        pl.BlockSpec((BLK, 5), lambda i: (i, 0)),
        pl.BlockSpec((BLK, 16), lambda i: (i, 0)),
        pl.BlockSpec((6, 8), lambda i: (0, 0)),
        pl.BlockSpec((13, H), lambda i: (0, 0)),
    ],
    out_specs=pl.BlockSpec((BLK, H), lambda i: (i, 0)),
    out_shape=jax.ShapeDtypeStruct((NPAD, H), jnp.float32),
)

_al_call = pl.pallas_call(
    _al_body,
    grid=(NBLK,),
    in_specs=[
        pl.BlockSpec((BLK, H), lambda i: (i, 0)),
        pl.BlockSpec((BLK, H), lambda i: (i, 0)),
        pl.BlockSpec((BLK, 16), lambda i: (i, 0)),
        pl.BlockSpec((1, H), lambda i: (0, 0)),
        pl.BlockSpec((H, H), lambda i: (0, 0)),
    ],
    out_specs=pl.BlockSpec((BLK, H), lambda i: (i, 0)),
    out_shape=jax.ShapeDtypeStruct((NPAD, H), jnp.float32),
)

_a3_call = pl.pallas_call(
    _a3_body,
    grid=(NBLK,),
    in_specs=[
        pl.BlockSpec((BLK, H), lambda i: (i, 0)),
        pl.BlockSpec((BLK, H), lambda i: (i, 0)),
        pl.BlockSpec((BLK, 16), lambda i: (i, 0)),
        pl.BlockSpec((1, H), lambda i: (0, 0)),
    ],
    out_specs=(pl.BlockSpec((BLK, H), lambda i: (i, 0)),
               pl.BlockSpec((1, 1, H), lambda i: (i, 0, 0))),
    out_shape=(jax.ShapeDtypeStruct((NPAD, H), jnp.float32),
               jax.ShapeDtypeStruct((NBLK, 1, H), jnp.float32)),
)

_f_call = pl.pallas_call(
    _f_body,
    out_shape=jax.ShapeDtypeStruct((C, 1), jnp.float32),
)


def kernel(kind_ids, other_feats, edge_index, cand_u, cand_v, kind_table,
           W0, b0, W1, b1, W2, b2, Wg, bg, Wc1, bc1, Wc2, bc2):
    src = edge_index[0].astype(jnp.int32)
    dst = edge_index[1].astype(jnp.int32)
    srcp = jnp.concatenate([src, jnp.zeros((EPAD - E,), jnp.int32)])
    dstp = jnp.concatenate([dst, jnp.full((EPAD - E,), NPAD, jnp.int32)])
    e3d = jnp.stack([srcp.reshape(EPAD // IDXN, IDXN),
                     dstp.reshape(EPAD // IDXN, IDXN)], axis=1)
    kidp = jnp.concatenate([kind_ids.astype(jnp.int32),
                            jnp.zeros((NPAD - N,), jnp.int32)]).reshape(NPAD, 1)
    ofp = jnp.concatenate([other_feats,
                           jnp.zeros((NPAD - N, 5), jnp.float32)])

    deg16 = _deg_kernel(e3d)
    xn0 = _a0_call(kidp, ofp, deg16, kind_table, W0)
    s0 = _edge_kernel(xn0, e3d)
    xn1 = _al_call(s0, xn0, deg16, b0.reshape(1, H), W1)
    s1 = _edge_kernel(xn1, e3d)
    xn2 = _al_call(s1, xn1, deg16, b1.reshape(1, H), W2)
    s2 = _edge_kernel(xn2, e3d)
    h3, psum = _a3_call(s2, xn2, deg16, b2.reshape(1, H))
    u_emb, v_emb = _cand_kernel(h3, cand_u.astype(jnp.int32),
                                cand_v.astype(jnp.int32))
    logits = _f_call(psum, Wg, bg.reshape(1, H), u_emb, v_emb,
                     Wc1, bc1.reshape(1, H), Wc2, bc2.reshape(1, 1))
    return logits[:, 0]


# speedup vs baseline: 10.2112x; 1.1649x over previous
import functools

import jax
import jax.numpy as jnp
from jax import lax
from jax.experimental import pallas as pl
from jax.experimental.pallas import tpu as pltpu
from jax.experimental.pallas import tpu_sc as plsc

N = 50000
E = 800000
C = 4096
H = 64

NC = 2
NS = 16
HALF = 25008
NPAD = 50176
ROWS_PT = HALF // NS
CHUNK = 128
IDXN = 112
K = 2
GROUP = K * IDXN
NB = 2
NG = 224
EPT = NG * GROUP
EPAD = EPT * NS
ZTAIL = ROWS_PT - (ROWS_PT // CHUNK) * CHUNK
BLK = 1024
NBLK = NPAD // BLK

_mesh = plsc.VectorSubcoreMesh(core_axis_name="c", subcore_axis_name="s")


def _fill_const(ref, rows, width, value):
    def body(r, _):
        for j in range(width // 16):
            ref[r, pl.ds(j * 16, 16)] = jnp.full((16,), value, jnp.float32)
        return 0
    lax.fori_loop(0, rows, body, 0)


def _zero_acc(acc, zsrc, s):
    def zbody(k, _):
        pltpu.sync_copy(zsrc, acc.at[pl.ds(s * ROWS_PT + k * CHUNK, CHUNK)])
        return 0
    lax.fori_loop(0, ROWS_PT // CHUNK, zbody, 0)
    pltpu.sync_copy(zsrc.at[pl.ds(0, ZTAIL)],
                    acc.at[pl.ds(s * ROWS_PT + (ROWS_PT // CHUNK) * CHUNK, ZTAIL)])

    @pl.when(s == 0)
    def _():
        pltpu.sync_copy(zsrc.at[pl.ds(0, 1)], acc.at[pl.ds(HALF, 1)])


def _compute_lidx(eb, lidx, base):
    for k in range(K):
        for j in range(IDXN // 16):
            d = eb[k, 1, pl.ds(j * 16, 16)]
            l = d - base
            bad = (l < 0) | (l >= HALF)
            lidx[k, pl.ds(j * 16, 16)] = jnp.where(bad, HALF, l)


@functools.partial(
    pl.kernel,
    out_type=jax.ShapeDtypeStruct((32, 65536), jnp.float32),
    mesh=_mesh,
    compiler_params=pltpu.CompilerParams(use_tc_tiling_on_sc=False,
                                         needs_layout_passes=False),
    scratch_types=[
        pltpu.VMEM((65536,), jnp.float32),
        pltpu.VMEM((2 * K * IDXN,), jnp.int32),
        pltpu.VMEM((2 * K * IDXN,), jnp.int32),
        pltpu.SemaphoreType.DMA,
        pltpu.SemaphoreType.DMA,
    ],
)
def _deg_kernel(e_hbm, out_hbm, hist, eb0, eb1, si0, si1):
    c = lax.axis_index("c")
    s = lax.axis_index("s")
    EB = (eb0, eb1)
    SI = (si0, si1)
    GWORDS = 2 * K * IDXN

    def zbody(r, _):
        hist[pl.ds(r * 16, 16)] = jnp.zeros((16,), jnp.float32)
        return 0
    lax.fori_loop(0, 65536 // 16, zbody, 0)

    wid = s * NC + c
    ngd = NG // 2
    word_base = wid * ngd * GWORDS
    ones16 = jnp.full((16,), 1.0, jnp.float32)

    def issue_idx(g, b):
        pltpu.async_copy(e_hbm.at[pl.ds(word_base + g * GWORDS, GWORDS)],
                         EB[b], SI[b])

    def drain_idx(b):
        pltpu.make_async_copy(e_hbm.at[pl.ds(0, GWORDS)], EB[b], SI[b]).wait()

    issue_idx(0, 0)

    def step(t, _):
        for b in range(NB):
            g = t * NB + b
            bp = 1 - b
            if b == NB - 1:
                @pl.when(t < ngd // NB - 1)
                def _():
                    issue_idx(g + 1, bp)
            else:
                issue_idx(g + 1, bp)
            drain_idx(b)
            for k in range(K):
                for j in range(IDXN // 16):
                    d = EB[b][pl.ds(k * 2 * IDXN + IDXN + j * 16, 16)]
                    plsc.addupdate_scatter(hist, [d], ones16)
        return 0
    lax.fori_loop(0, ngd // NB, step, 0)
    pltpu.sync_copy(hist, out_hbm.at[wid])


def _dsum_body(h_ref, out_ref):
    out_ref[...] = jnp.sum(h_ref[...], axis=0)


_dsum_call = pl.pallas_call(
    _dsum_body,
    grid=(4,),
    in_specs=[pl.BlockSpec((32, 128, 128), lambda i: (0, i, 0))],
    out_specs=pl.BlockSpec((128, 128), lambda i: (i, 0)),
    out_shape=jax.ShapeDtypeStruct((512, 128), jnp.float32),
)


@functools.partial(
    pl.kernel,
    out_type=jax.ShapeDtypeStruct((NPAD, H), jnp.float32),
    mesh=_mesh,
    compiler_params=pltpu.CompilerParams(use_tc_tiling_on_sc=False),
    scratch_types=[
        pltpu.VMEM_SHARED((HALF + 1, H), jnp.float32),
        pltpu.VMEM((K, 2, IDXN), jnp.int32),
        pltpu.VMEM((K, 2, IDXN), jnp.int32),
        pltpu.VMEM((K, IDXN), jnp.int32),
        pltpu.VMEM((K, IDXN), jnp.int32),
        pltpu.VMEM((GROUP, H), jnp.float32),
        pltpu.VMEM((GROUP, H), jnp.float32),
        pltpu.SemaphoreType.DMA,
        pltpu.SemaphoreType.DMA,
        pltpu.SemaphoreType.DMA,
        pltpu.SemaphoreType.DMA,
        pltpu.SemaphoreType.DMA,
        pltpu.SemaphoreType.DMA,
    ],
)
def _edge_kernel(xn_hbm, e_hbm, out_hbm, acc,
                 eb0, eb1, li0, li1, r0, r1,
                 si0, si1, sg0, sg1, ss0, ss1):
    c = lax.axis_index("c")
    s = lax.axis_index("s")
    base = c * HALF
    EB = (eb0, eb1)
    LI = (li0, li1)
    R = (r0, r1)
    SI = (si0, si1)
    SG = (sg0, sg1)
    SS = (ss0, ss1)

    _fill_const(r0, CHUNK, H, 0.0)
    _zero_acc(acc, r0.at[pl.ds(0, CHUNK)], s)
    plsc.subcore_barrier()

    row_base = s * (K * NG)

    def issue_idx(g, b):
        pltpu.async_copy(e_hbm.at[pl.ds(row_base + K * g, K)], EB[b], SI[b])

    def drain_idx(b):
        pltpu.make_async_copy(e_hbm.at[pl.ds(0, K)], EB[b], SI[b]).wait()

    def issue_gather(b):
        for k in range(K):
            pltpu.async_copy(xn_hbm.at[EB[b].at[k, 0]],
                             R[b].at[pl.ds(k * IDXN, IDXN)], SG[b])

    def drain_gather(b):
        pltpu.make_async_copy(xn_hbm.at[pl.ds(0, GROUP)], R[b], SG[b]).wait()

    def issue_scatter(b):
        for k in range(K):
            pltpu.async_copy(R[b].at[pl.ds(k * IDXN, IDXN)],
                             acc.at[LI[b].at[k]], SS[b], add=True)

    def drain_scatter(b):
        pltpu.make_async_copy(xn_hbm.at[pl.ds(0, GROUP)], R[b], SS[b]).wait()

    issue_idx(0, 0)

    def step(t, _):
        for b in range(NB):
            g = t * NB + b
            bp = 1 - b
            drain_idx(b)

            @pl.when(t >= 1)
            def _():
                drain_scatter(b)
            issue_gather(b)
            _compute_lidx(EB[b], LI[b], base)
            if b == 0:
                @pl.when(t >= 1)
                def _():
                    drain_gather(bp)
                    issue_scatter(bp)
            else:
                drain_gather(bp)
                issue_scatter(bp)
            if b == NB - 1:
                @pl.when(t < NG // NB - 1)
                def _():
                    issue_idx(g + 1, bp)
            else:
                issue_idx(g + 1, bp)
        return 0
    lax.fori_loop(0, NG // NB, step, 0)
    drain_gather((NG - 1) % NB)
    issue_scatter((NG - 1) % NB)
    for b in range(NB):
        drain_scatter(b)
    plsc.subcore_barrier()
    pltpu.sync_copy(acc.at[pl.ds(s * ROWS_PT, ROWS_PT)],
                    out_hbm.at[pl.ds(base + s * ROWS_PT, ROWS_PT)])


@functools.partial(
    pl.kernel,
    out_type=(jax.ShapeDtypeStruct((C, H), jnp.float32),
              jax.ShapeDtypeStruct((C, H), jnp.float32)),
    mesh=_mesh,
    compiler_params=pltpu.CompilerParams(use_tc_tiling_on_sc=False),
    scratch_types=[
        pltpu.VMEM((CHUNK,), jnp.int32),
        pltpu.VMEM((CHUNK, H), jnp.float32),
        pltpu.SemaphoreType.DMA,
    ],
)
def _cand_kernel(h_hbm, cu_hbm, cv_hbm, u_out, v_out, idx, rows, sem):
    c = lax.axis_index("c")
    s = lax.axis_index("s")
    b = (s * NC + c) * CHUNK
    pltpu.sync_copy(cu_hbm.at[pl.ds(b, CHUNK)], idx)
    pltpu.async_copy(h_hbm.at[idx], rows, sem).wait()
    pltpu.sync_copy(rows, u_out.at[pl.ds(b, CHUNK)])
    pltpu.sync_copy(cv_hbm.at[pl.ds(b, CHUNK)], idx)
    pltpu.async_copy(h_hbm.at[idx], rows, sem).wait()
    pltpu.sync_copy(rows, v_out.at[pl.ds(b, CHUNK)])


def _a0_body(kid_ref, of_ref, deg_ref, kt_ref, w0_ref, out_ref):
    norm = lax.rsqrt(deg_ref[:, 0:1] + 1.0)
    iot = lax.broadcasted_iota(jnp.int32, (1, 8), 1)
    onehot = jnp.where(kid_ref[...] == iot, 1.0, 0.0)
    t0 = jnp.dot(kt_ref[...], w0_ref[0:8, :], preferred_element_type=jnp.float32)
    x = (jnp.dot(onehot[:, 0:6], t0, preferred_element_type=jnp.float32)
         + jnp.dot(of_ref[...], w0_ref[8:13, :], preferred_element_type=jnp.float32))
    out_ref[...] = x * norm


def _al_body(s_ref, xn_ref, deg_ref, b_ref, w_ref, out_ref):
    norm = lax.rsqrt(deg_ref[:, 0:1] + 1.0)
    h = jnp.maximum(norm * (s_ref[...] + xn_ref[...]) + b_ref[...], 0.0)
    out_ref[...] = jnp.dot(h, w_ref[...], preferred_element_type=jnp.float32) * norm


def _a3_body(s_ref, xn_ref, deg_ref, b_ref, h_out, psum_out):
    i = pl.program_id(0)
    norm = lax.rsqrt(deg_ref[:, 0:1] + 1.0)
    h = jnp.maximum(norm * (s_ref[...] + xn_ref[...]) + b_ref[...], 0.0)
    h_out[...] = h
    row = i * BLK + lax.broadcasted_iota(jnp.int32, (BLK, 1), 0)
    psum_out[...] = jnp.sum(jnp.where(row < N, h, 0.0), axis=0,
                            keepdims=True)[None, :, :]


def _f_body(ps_ref, wg_ref, bg_ref, u_ref, v_ref, wc1_ref, bc1_ref,
            wc2_ref, bc2_ref, out_ref):
    hsum = jnp.sum(ps_ref[...], axis=0)
    g = jnp.dot(hsum * (1.0 / N), wg_ref[...],
                preferred_element_type=jnp.float32) + bg_ref[...]
    cvec = jnp.dot(g, wc1_ref[2 * H:3 * H, :],
                   preferred_element_type=jnp.float32) + bc1_ref[...]
    hid = jnp.maximum(
        jnp.dot(u_ref[...], wc1_ref[0:H, :], preferred_element_type=jnp.float32)
        + jnp.dot(v_ref[...], wc1_ref[H:2 * H, :], preferred_element_type=jnp.float32)
        + cvec, 0.0)
    out_ref[...] = jnp.dot(hid, wc2_ref[...],
                           preferred_element_type=jnp.float32) + bc2_ref[...]


_a0_call = pl.pallas_call(
    _a0_body,
    grid=(NBLK,),
    in_specs=[
        pl.BlockSpec((BLK, 1), lambda i: (i, 0)),
        pl.BlockSpec((BLK, 5), lambda i: (i, 0)),
        pl.BlockSpec((BLK, 1), lambda i: (i, 0)),
        pl.BlockSpec((6, 8), lambda i: (0, 0)),
        pl.BlockSpec((13, H), lambda i: (0, 0)),
    ],
    out_specs=pl.BlockSpec((BLK, H), lambda i: (i, 0)),
    out_shape=jax.ShapeDtypeStruct((NPAD, H), jnp.float32),
)

_al_call = pl.pallas_call(
    _al_body,
    grid=(NBLK,),
    in_specs=[
        pl.BlockSpec((BLK, H), lambda i: (i, 0)),
        pl.BlockSpec((BLK, H), lambda i: (i, 0)),
        pl.BlockSpec((BLK, 1), lambda i: (i, 0)),
        pl.BlockSpec((1, H), lambda i: (0, 0)),
        pl.BlockSpec((H, H), lambda i: (0, 0)),
    ],
    out_specs=pl.BlockSpec((BLK, H), lambda i: (i, 0)),
    out_shape=jax.ShapeDtypeStruct((NPAD, H), jnp.float32),
)

_a3_call = pl.pallas_call(
    _a3_body,
    grid=(NBLK,),
    in_specs=[
        pl.BlockSpec((BLK, H), lambda i: (i, 0)),
        pl.BlockSpec((BLK, H), lambda i: (i, 0)),
        pl.BlockSpec((BLK, 1), lambda i: (i, 0)),
        pl.BlockSpec((1, H), lambda i: (0, 0)),
    ],
    out_specs=(pl.BlockSpec((BLK, H), lambda i: (i, 0)),
               pl.BlockSpec((1, 1, H), lambda i: (i, 0, 0))),
    out_shape=(jax.ShapeDtypeStruct((NPAD, H), jnp.float32),
               jax.ShapeDtypeStruct((NBLK, 1, H), jnp.float32)),
)

_f_call = pl.pallas_call(
    _f_body,
    out_shape=jax.ShapeDtypeStruct((C, 1), jnp.float32),
)


def kernel(kind_ids, other_feats, edge_index, cand_u, cand_v, kind_table,
           W0, b0, W1, b1, W2, b2, Wg, bg, Wc1, bc1, Wc2, bc2):
    src = edge_index[0].astype(jnp.int32)
    dst = edge_index[1].astype(jnp.int32)
    srcp = jnp.concatenate([src, jnp.zeros((EPAD - E,), jnp.int32)])
    dstp = jnp.concatenate([dst, jnp.full((EPAD - E,), NPAD, jnp.int32)])
    e3d = jnp.stack([srcp.reshape(EPAD // IDXN, IDXN),
                     dstp.reshape(EPAD // IDXN, IDXN)], axis=1)
    kidp = jnp.concatenate([kind_ids.astype(jnp.int32),
                            jnp.zeros((NPAD - N,), jnp.int32)]).reshape(NPAD, 1)
    ofp = jnp.concatenate([other_feats,
                           jnp.zeros((NPAD - N, 5), jnp.float32)])

    hists = _deg_kernel(e3d.reshape(-1))
    degflat = _dsum_call(hists.reshape(32, 512, 128))
    deg1 = degflat.reshape(65536)[:NPAD].reshape(NPAD, 1)
    xn0 = _a0_call(kidp, ofp, deg1, kind_table, W0)
    s0 = _edge_kernel(xn0, e3d)
    xn1 = _al_call(s0, xn0, deg1, b0.reshape(1, H), W1)
    s1 = _edge_kernel(xn1, e3d)
    xn2 = _al_call(s1, xn1, deg1, b1.reshape(1, H), W2)
    s2 = _edge_kernel(xn2, e3d)
    h3, psum = _a3_call(s2, xn2, deg1, b2.reshape(1, H))
    u_emb, v_emb = _cand_kernel(h3, cand_u.astype(jnp.int32),
                                cand_v.astype(jnp.int32))
    logits = _f_call(psum, Wg, bg.reshape(1, H), u_emb, v_emb,
                     Wc1, bc1.reshape(1, H), Wc2, bc2.reshape(1, 1))
    return logits[:, 0]
